# Initial kernel scaffold; baseline (speedup 1.0000x reference)
import jax
import jax.numpy as jnp
from jax.experimental import pallas as pl


def kernel(z, t, edge_index, edge_weight, edge_vec_norm, edge_attr, node_attr, emb_table, mix1_W, mix1_b, mix2_W, mix2_b, emb2_W, emb2_b, dp1_W, dp1_b, dp2_W, dp2_b, dp3_W, dp3_b, lt0_W, lt1_W, lt2_W, ls0_W, ls0_b, ls1_W, ls1_b, ln_g, ln_b):
    raise NotImplementedError("write your pallas kernel here")


# speedup vs baseline: 14.8173x; 14.8173x over previous
import functools

import jax
import jax.numpy as jnp
from jax import lax
from jax.experimental import pallas as pl
from jax.experimental.pallas import tpu as pltpu

N = 10000
E = 160000
H = 64
RBF = 32
CUTOFF_UPPER = 5.0

EB = 1000
NBLK = E // EB


def _silu(x):
    return x * jax.nn.sigmoid(x)


def _node_prep_body(z_ref, t_ref, m1p_ref, m1b_ref,
                    m2_ref, m2b_ref, e2a_ref, e2b_ref, e2bias_ref,
                    p_ref, q_ref):
    z = z_ref[...]
    onehot = (z == lax.broadcasted_iota(jnp.int32, (1, 128), 1)).astype(jnp.float32)
    Z = jnp.dot(onehot, m1p_ref[...][:128, :], preferred_element_type=jnp.float32,
                precision=lax.Precision.HIGHEST)
    t = t_ref[...]
    zc = jnp.concatenate([Z, t, jnp.zeros((Z.shape[0], 63), jnp.float32)], axis=1)
    pre = jnp.dot(zc, m1p_ref[...][128:, :], preferred_element_type=jnp.float32) \
        + m1b_ref[...]
    h1 = _silu(pre)
    zm = jnp.dot(h1, m2_ref[...], preferred_element_type=jnp.float32) + m2b_ref[...]
    p_ref[...] = jnp.dot(zm, e2a_ref[...], preferred_element_type=jnp.float32)
    q_ref[...] = (jnp.dot(zm, e2b_ref[...], preferred_element_type=jnp.float32)
                  + e2bias_ref[...])


def _node_prep(z, t, emb_table, mix1_W, mix1_b, mix2_W, mix2_b, emb2_W, emb2_b):
    emb_pad = jnp.concatenate([emb_table,
                               jnp.zeros((128 - emb_table.shape[0], H),
                                         jnp.float32)], axis=0)
    m1pad = jnp.concatenate([mix1_W.T, jnp.zeros((63, H), jnp.float32)], axis=0)
    m1p = jnp.concatenate([emb_pad, m1pad], axis=0)
    return pl.pallas_call(
        _node_prep_body,
        out_shape=(jax.ShapeDtypeStruct((N, H), jnp.float32),
                   jax.ShapeDtypeStruct((N, H), jnp.float32)),
    )(z.reshape(N, 1).astype(jnp.int32), t, m1p,
      mix1_b[None, :], mix2_W.T, mix2_b[None, :],
      emb2_W[:, :H].T, emb2_W[:, H:].T, emb2_b[None, :])


def _edge_body(idx_ref, ew_ref, evn_ref, ea_ref, p_ref, q_ref,
               dp1_ref, dp2_ref, dp3_ref, db_ref,
               w_ref, zc_ref, pay_ref):
    @pl.when(pl.program_id(0) == 0)
    def _init():
        w_ref[...] = jnp.zeros_like(w_ref)

    def gather(i, _):
        s = idx_ref[0, 0, i]
        d = idx_ref[0, 1, i]
        zc_ref[pl.ds(i, 1), :] = p_ref[pl.ds(s, 1), :] + q_ref[pl.ds(d, 1), :]
        return 0
    lax.fori_loop(0, EB, gather, 0, unroll=False)

    ew = ew_ref[...]
    c = 0.5 * (jnp.cos(ew * (jnp.pi / CUTOFF_UPPER)) + 1.0)
    c = c * (ew < CUTOFF_UPPER).astype(jnp.float32)
    ea = ea_ref[...]
    zc = zc_ref[...] * c
    d1 = (jnp.dot(ea, dp1_ref[...], preferred_element_type=jnp.float32)
          + db_ref[0, :][None, :]) * zc
    d2 = (jnp.dot(ea, dp2_ref[...], preferred_element_type=jnp.float32)
          + db_ref[1, :][None, :]) * zc
    d3 = (jnp.dot(ea, dp3_ref[...], preferred_element_type=jnp.float32)
          + db_ref[2, :][None, :]) * zc
    v = evn_ref[...]
    vx, vy, vz = v[:, 0:1], v[:, 1:2], v[:, 2:3]
    m = (vx * vx + vy * vy + vz * vz) * (1.0 / 3.0)
    pay_ref[...] = jnp.concatenate(
        [d1,
         d2 * vx, d2 * vy, d2 * vz,
         d3 * (vx * vx - m), d3 * (vy * vy - m), d3 * (vz * vz - m),
         d3 * (vx * vy), d3 * (vx * vz), d3 * (vy * vz)], axis=1)

    def scatter(i, _):
        s = idx_ref[0, 0, i]
        w_ref[pl.ds(s, 1), :] += pay_ref[pl.ds(i, 1), :]
        return 0
    lax.fori_loop(0, EB, scatter, 0, unroll=False)


def _edge_stage(edge_index, edge_weight, edge_vec_norm, edge_attr, p, q,
                dp1_W, dp1_b, dp2_W, dp2_b, dp3_W, dp3_b):
    idx3 = edge_index.astype(jnp.int32).reshape(2, NBLK, EB).transpose(1, 0, 2)
    db = jnp.stack([dp1_b, dp2_b, dp3_b], axis=0)
    full = lambda shape: pl.BlockSpec(shape, lambda i: (0,) * len(shape))
    return pl.pallas_call(
        _edge_body,
        grid=(NBLK,),
        in_specs=[
            pl.BlockSpec((1, 2, EB), lambda i: (i, 0, 0), memory_space=pltpu.SMEM),
            pl.BlockSpec((EB, 1), lambda i: (i, 0)),
            pl.BlockSpec((EB, 3), lambda i: (i, 0)),
            pl.BlockSpec((EB, RBF), lambda i: (i, 0)),
            full((N, H)), full((N, H)),
            full((RBF, H)), full((RBF, H)), full((RBF, H)), full((3, H)),
        ],
        out_specs=pl.BlockSpec((N, 10 * H), lambda i: (0, 0)),
        out_shape=jax.ShapeDtypeStruct((N, 10 * H), jnp.float32),
        scratch_shapes=[pltpu.VMEM((EB, H), jnp.float32),
                        pltpu.VMEM((EB, 10 * H), jnp.float32)],
    )(idx3, edge_weight.reshape(E, 1), edge_vec_norm, edge_attr, p, q,
      dp1_W.T, dp2_W.T, dp3_W.T, db)


NB3 = 2000


def _final_body(w_ref, lng_ref, lnb_ref, ls0_ref, ls0b_ref,
                ls1a_ref, ls1b_ref, ls1c_ref, ls1bias_ref,
                lt0_ref, lt1_ref, lt2_ref, o_ref):
    w = w_ref[...]
    s1 = w[:, 0:H]
    w2 = [w[:, H * (1 + i):H * (2 + i)] for i in range(3)]
    w3 = [w[:, H * (4 + i):H * (5 + i)] for i in range(6)]
    nrm = (3.0 * s1 * s1
           + 2.0 * (w2[0] * w2[0] + w2[1] * w2[1] + w2[2] * w2[2])
           + w3[0] * w3[0] + w3[1] * w3[1] + w3[2] * w3[2]
           + 2.0 * (w3[3] * w3[3] + w3[4] * w3[4] + w3[5] * w3[5]))
    nrm = jnp.maximum(nrm, 0.01)
    mu = jnp.mean(nrm, axis=-1, keepdims=True)
    var = jnp.mean((nrm - mu) ** 2, axis=-1, keepdims=True)
    nrm = (nrm - mu) / jnp.sqrt(var + 1e-5) * lng_ref[...] + lnb_ref[...]
    f = _silu(jnp.dot(nrm, ls0_ref[...], preferred_element_type=jnp.float32)
              + ls0b_ref[...])
    n0 = _silu(jnp.dot(f, ls1a_ref[...], preferred_element_type=jnp.float32)
               + ls1bias_ref[0, :][None, :])
    n1 = _silu(jnp.dot(f, ls1b_ref[...], preferred_element_type=jnp.float32)
               + ls1bias_ref[1, :][None, :])
    n2 = _silu(jnp.dot(f, ls1c_ref[...], preferred_element_type=jnp.float32)
               + ls1bias_ref[2, :][None, :])
    ip = jnp.dot(s1, lt0_ref[...], preferred_element_type=jnp.float32) * n0
    w2p = [jnp.dot(w2[i], lt1_ref[...], preferred_element_type=jnp.float32) * n1
           for i in range(3)]
    w3p = [jnp.dot(w3[i], lt2_ref[...], preferred_element_type=jnp.float32) * n2
           for i in range(6)]
    o_ref[0] = ip + w3p[0]
    o_ref[1] = -w2p[2] + w3p[3]
    o_ref[2] = w2p[1] + w3p[4]
    o_ref[3] = w2p[2] + w3p[3]
    o_ref[4] = ip + w3p[1]
    o_ref[5] = -w2p[0] + w3p[5]
    o_ref[6] = -w2p[1] + w3p[4]
    o_ref[7] = w2p[0] + w3p[5]
    o_ref[8] = ip + w3p[2]


def _node_final(w, ln_g, ln_b, ls0_W, ls0_b, ls1_W, ls1_b, lt0_W, lt1_W, lt2_W):
    ls1bias = jnp.stack([ls1_b[0::3], ls1_b[1::3], ls1_b[2::3]], axis=0)
    full = lambda shape: pl.BlockSpec(shape, lambda i: (0,) * len(shape))
    out = pl.pallas_call(
        _final_body,
        grid=(N // NB3,),
        in_specs=[
            pl.BlockSpec((NB3, 10 * H), lambda i: (i, 0)),
            full((1, H)), full((1, H)),
            full((H, 2 * H)), full((1, 2 * H)),
            full((2 * H, H)), full((2 * H, H)), full((2 * H, H)), full((3, H)),
            full((H, H)), full((H, H)), full((H, H)),
        ],
        out_specs=pl.BlockSpec((9, NB3, H), lambda i: (0, i, 0)),
        out_shape=jax.ShapeDtypeStruct((9, N, H), jnp.float32),
    )(w, ln_g[None, :], ln_b[None, :], ls0_W.T, ls0_b[None, :],
      ls1_W[0::3, :].T, ls1_W[1::3, :].T, ls1_W[2::3, :].T, ls1bias,
      lt0_W.T, lt1_W.T, lt2_W.T)
    return out


@jax.jit
def kernel(z, t, edge_index, edge_weight, edge_vec_norm, edge_attr, node_attr,
           emb_table, mix1_W, mix1_b, mix2_W, mix2_b, emb2_W, emb2_b,
           dp1_W, dp1_b, dp2_W, dp2_b, dp3_W, dp3_b,
           lt0_W, lt1_W, lt2_W, ls0_W, ls0_b, ls1_W, ls1_b, ln_g, ln_b):
    p, q = _node_prep(z, t, emb_table, mix1_W, mix1_b, mix2_W, mix2_b,
                      emb2_W, emb2_b)
    w = _edge_stage(edge_index, edge_weight, edge_vec_norm, edge_attr, p, q,
                    dp1_W, dp1_b, dp2_W, dp2_b, dp3_W, dp3_b)
    planes = _node_final(w, ln_g, ln_b, ls0_W, ls0_b, ls1_W, ls1_b,
                         lt0_W, lt1_W, lt2_W)
    return planes.transpose(1, 2, 0).reshape(N, H, 3, 3)


# speedup vs baseline: 15.2359x; 1.0283x over previous
import functools

import jax
import jax.numpy as jnp
from jax import lax
from jax.experimental import pallas as pl
from jax.experimental.pallas import tpu as pltpu, tpu_sc as plsc

N = 10000
E = 160000
H = 64
RBF = 32
CUTOFF_UPPER = 5.0


def _silu(x):
    return x * jax.nn.sigmoid(x)


def _node_prep_body(z_ref, t_ref, m1p_ref, m1b_ref,
                    m2_ref, m2b_ref, e2a_ref, e2b_ref, e2bias_ref,
                    p_ref):
    z = z_ref[...]
    onehot = (z == lax.broadcasted_iota(jnp.int32, (1, 128), 1)).astype(jnp.float32)
    Z = jnp.dot(onehot, m1p_ref[...][:128, :], preferred_element_type=jnp.float32,
                precision=lax.Precision.HIGHEST)
    t = t_ref[...]
    zc = jnp.concatenate([Z, t, jnp.zeros((Z.shape[0], 63), jnp.float32)], axis=1)
    pre = jnp.dot(zc, m1p_ref[...][128:, :], preferred_element_type=jnp.float32) \
        + m1b_ref[...]
    h1 = _silu(pre)
    zm = jnp.dot(h1, m2_ref[...], preferred_element_type=jnp.float32) + m2b_ref[...]
    p_ref[:, :H] = jnp.dot(zm, e2a_ref[...], preferred_element_type=jnp.float32)
    p_ref[:, H:] = (jnp.dot(zm, e2b_ref[...], preferred_element_type=jnp.float32)
                    + e2bias_ref[...])


def _node_prep(z, t, emb_table, mix1_W, mix1_b, mix2_W, mix2_b, emb2_W, emb2_b):
    emb_pad = jnp.concatenate([emb_table,
                               jnp.zeros((128 - emb_table.shape[0], H),
                                         jnp.float32)], axis=0)
    m1pad = jnp.concatenate([mix1_W.T, jnp.zeros((63, H), jnp.float32)], axis=0)
    m1p = jnp.concatenate([emb_pad, m1pad], axis=0)
    return pl.pallas_call(
        _node_prep_body,
        out_shape=jax.ShapeDtypeStruct((N, 2 * H), jnp.float32),
    )(z.reshape(N, 1).astype(jnp.int32), t, m1p,
      mix1_b[None, :], mix2_W.T, mix2_b[None, :],
      emb2_W[:, :H].T, emb2_W[:, H:].T, emb2_b[None, :])


EB = 2000
NBLK = E // EB


def _coef_body(ew_ref, evn_ref, ea_ref, dp1_ref, dp2_ref, dp3_ref, db_ref,
               a_ref, a2_ref, a3_ref, g_ref):
    ew = ew_ref[...]
    c = 0.5 * (jnp.cos(ew * (jnp.pi / CUTOFF_UPPER)) + 1.0)
    c = c * (ew < CUTOFF_UPPER).astype(jnp.float32)
    ea = ea_ref[...]
    a1 = (jnp.dot(ea, dp1_ref[...], preferred_element_type=jnp.float32)
          + db_ref[0, :][None, :]) * c
    a2 = (jnp.dot(ea, dp2_ref[...], preferred_element_type=jnp.float32)
          + db_ref[1, :][None, :]) * c
    a3 = (jnp.dot(ea, dp3_ref[...], preferred_element_type=jnp.float32)
          + db_ref[2, :][None, :]) * c
    a_ref[...] = a1
    a2_ref[...] = a2
    a3_ref[...] = a3
    v = evn_ref[...]
    vx, vy, vz = v[:, 0:1], v[:, 1:2], v[:, 2:3]
    m = (vx * vx + vy * vy + vz * vz) * (1.0 / 3.0)
    g_ref[...] = jnp.concatenate(
        [vx, vy, vz, vx * vx - m, vy * vy - m, vz * vz - m,
         vx * vy, vx * vz, vy * vz,
         jnp.zeros((vx.shape[0], 7), jnp.float32)], axis=1)


def _edge_coef(edge_weight, edge_vec_norm, edge_attr,
               dp1_W, dp1_b, dp2_W, dp2_b, dp3_W, dp3_b):
    db = jnp.stack([dp1_b, dp2_b, dp3_b], axis=0)
    full = lambda shape: pl.BlockSpec(shape, lambda i: (0,) * len(shape))
    return pl.pallas_call(
        _coef_body,
        grid=(NBLK,),
        in_specs=[
            pl.BlockSpec((EB, 1), lambda i: (i, 0)),
            pl.BlockSpec((EB, 3), lambda i: (i, 0)),
            pl.BlockSpec((EB, RBF), lambda i: (i, 0)),
            full((RBF, H)), full((RBF, H)), full((RBF, H)), full((3, H)),
        ],
        out_specs=(pl.BlockSpec((EB, H), lambda i: (i, 0)),
                   pl.BlockSpec((EB, H), lambda i: (i, 0)),
                   pl.BlockSpec((EB, H), lambda i: (i, 0)),
                   pl.BlockSpec((EB, 16), lambda i: (i, 0))),
        out_shape=(jax.ShapeDtypeStruct((E, H), jnp.float32),
                   jax.ShapeDtypeStruct((E, H), jnp.float32),
                   jax.ShapeDtypeStruct((E, H), jnp.float32),
                   jax.ShapeDtypeStruct((E, 16), jnp.float32)),
    )(edge_weight.reshape(E, 1), edge_vec_norm, edge_attr,
      dp1_W.T, dp2_W.T, dp3_W.T, db)


C = 40
EPT = E // 16
NCHUNK = EPT // C
SPLIT0 = 125
NPAD = 10240
ROWS_PT = NPAD // 16

BLOCK_AK = [0, 1, 1, 1, 2, 2, 2, 2, 2, 2]
SLOT_BLOCKS = [(0, 1), (2, 3), (4, 5), (6, 7), (8, 9)]

_mesh = plsc.VectorSubcoreMesh(core_axis_name="c", subcore_axis_name="s")


@functools.partial(
    pl.kernel, mesh=_mesh,
    out_type=jax.ShapeDtypeStruct((6, NPAD, 128), jnp.float32),
    scratch_types=[
        pltpu.VMEM((C,), jnp.int32),
        pltpu.VMEM((C,), jnp.int32),
        pltpu.VMEM((C, 2 * H), jnp.float32),
        pltpu.VMEM((C, 2 * H), jnp.float32),
        pltpu.VMEM((C, H), jnp.float32),
        pltpu.VMEM((C, H), jnp.float32),
        pltpu.VMEM((C, 16), jnp.float32),
        pltpu.VMEM((C, 2 * H), jnp.float32),
        pltpu.VMEM_SHARED((NPAD, 128), jnp.float32),
        pltpu.SemaphoreType.DMA,
    ],
)
def _sc_edge(srcs_hbm, dsts_hbm, pq_hbm, a1_hbm, a2_hbm, a3_hbm, g_hbm,
             zr_hbm, w_hbm,
             src_v, dst_v, p_v, q_v, a0_v, a1_v, g_v, pay_v, acc, sem):
    core = lax.axis_index("c")
    sid = lax.axis_index("s")

    ak_hbm = [a1_hbm, a2_hbm, a3_hbm]

    def run_chunks(k_lo, k_hi, blocks):
        k0, k1 = BLOCK_AK[blocks[0]], BLOCK_AK[blocks[1]]
        stages = [a0_v, a1_v if k1 != k0 else a0_v]

        def chunk(k, _):
            base = sid * EPT + k * C
            pltpu.sync_copy(srcs_hbm.at[pl.ds(base, C)], src_v)
            pltpu.sync_copy(dsts_hbm.at[pl.ds(base, C)], dst_v)
            pltpu.async_copy(pq_hbm.at[src_v], p_v, sem).wait()
            pltpu.async_copy(pq_hbm.at[dst_v], q_v, sem).wait()
            pltpu.sync_copy(ak_hbm[k0].at[pl.ds(base, C)], a0_v)
            if k1 != k0:
                pltpu.sync_copy(ak_hbm[k1].at[pl.ds(base, C)], a1_v)
            pltpu.sync_copy(g_hbm.at[pl.ds(base, C)], g_v)

            def edge(e, _):
                zs = [p_v[e, pl.ds(16 * hb, 16)] + q_v[e, pl.ds(64 + 16 * hb, 16)]
                      for hb in range(4)]
                for b, j in enumerate(blocks):
                    a_v = stages[b]
                    if j == 0:
                        for hb in range(4):
                            d = a_v[e, pl.ds(16 * hb, 16)] * zs[hb]
                            pay_v[e, pl.ds(64 * b + 16 * hb, 16)] = d
                    else:
                        g = g_v[e, :][j - 1]
                        for hb in range(4):
                            d = a_v[e, pl.ds(16 * hb, 16)] * zs[hb]
                            pay_v[e, pl.ds(64 * b + 16 * hb, 16)] = d * g
                return 0
            lax.fori_loop(0, C, edge, 0, unroll=False)
            pltpu.sync_copy(pay_v, acc.at[src_v], add=True)
            return 0
        lax.fori_loop(k_lo, k_hi, chunk, 0, unroll=False)

    for p in range(3):
        pltpu.sync_copy(zr_hbm, acc.at[pl.ds(sid * ROWS_PT, ROWS_PT)])
        plsc.subcore_barrier()
        if p < 2:
            bl0, bl1 = SLOT_BLOCKS[2 * p], SLOT_BLOCKS[2 * p + 1]

            @pl.when(core == 0)
            def _c0():
                run_chunks(0, NCHUNK, bl0)

            @pl.when(core == 1)
            def _c1():
                run_chunks(0, NCHUNK, bl1)
        else:
            bl = SLOT_BLOCKS[4]

            @pl.when(core == 0)
            def _c0():
                run_chunks(0, SPLIT0, bl)

            @pl.when(core == 1)
            def _c1():
                run_chunks(SPLIT0, NCHUNK, bl)

        plsc.subcore_barrier()
        pltpu.sync_copy(acc.at[pl.ds(sid * ROWS_PT, ROWS_PT)],
                        w_hbm.at[2 * p + core, pl.ds(sid * ROWS_PT, ROWS_PT)])
        plsc.subcore_barrier()


NB3 = 2000


def _final_body(w_ref, lng_ref, lnb_ref, ls0_ref, ls0b_ref,
                ls1a_ref, ls1b_ref, ls1c_ref, ls1bias_ref,
                lt0_ref, lt1_ref, lt2_ref, o_ref):
    w = w_ref[...]
    s1 = w[0, :, 0:H]
    w2 = [w[0, :, H:2 * H], w[1, :, 0:H], w[1, :, H:2 * H]]
    w3 = [w[2, :, 0:H], w[2, :, H:2 * H],
          w[3, :, 0:H], w[3, :, H:2 * H],
          w[4, :, 0:H] + w[5, :, 0:H], w[4, :, H:2 * H] + w[5, :, H:2 * H]]
    nrm = (3.0 * s1 * s1
           + 2.0 * (w2[0] * w2[0] + w2[1] * w2[1] + w2[2] * w2[2])
           + w3[0] * w3[0] + w3[1] * w3[1] + w3[2] * w3[2]
           + 2.0 * (w3[3] * w3[3] + w3[4] * w3[4] + w3[5] * w3[5]))
    nrm = jnp.maximum(nrm, 0.01)
    mu = jnp.mean(nrm, axis=-1, keepdims=True)
    var = jnp.mean((nrm - mu) ** 2, axis=-1, keepdims=True)
    nrm = (nrm - mu) / jnp.sqrt(var + 1e-5) * lng_ref[...] + lnb_ref[...]
    f = _silu(jnp.dot(nrm, ls0_ref[...], preferred_element_type=jnp.float32)
              + ls0b_ref[...])
    n0 = _silu(jnp.dot(f, ls1a_ref[...], preferred_element_type=jnp.float32)
               + ls1bias_ref[0, :][None, :])
    n1 = _silu(jnp.dot(f, ls1b_ref[...], preferred_element_type=jnp.float32)
               + ls1bias_ref[1, :][None, :])
    n2 = _silu(jnp.dot(f, ls1c_ref[...], preferred_element_type=jnp.float32)
               + ls1bias_ref[2, :][None, :])
    ip = jnp.dot(s1, lt0_ref[...], preferred_element_type=jnp.float32) * n0
    w2p = [jnp.dot(w2[i], lt1_ref[...], preferred_element_type=jnp.float32) * n1
           for i in range(3)]
    w3p = [jnp.dot(w3[i], lt2_ref[...], preferred_element_type=jnp.float32) * n2
           for i in range(6)]
    o_ref[0] = ip + w3p[0]
    o_ref[1] = -w2p[2] + w3p[3]
    o_ref[2] = w2p[1] + w3p[4]
    o_ref[3] = w2p[2] + w3p[3]
    o_ref[4] = ip + w3p[1]
    o_ref[5] = -w2p[0] + w3p[5]
    o_ref[6] = -w2p[1] + w3p[4]
    o_ref[7] = w2p[0] + w3p[5]
    o_ref[8] = ip + w3p[2]


def _node_final(w, ln_g, ln_b, ls0_W, ls0_b, ls1_W, ls1_b, lt0_W, lt1_W, lt2_W):
    ls1bias = jnp.stack([ls1_b[0::3], ls1_b[1::3], ls1_b[2::3]], axis=0)
    full = lambda shape: pl.BlockSpec(shape, lambda i: (0,) * len(shape))
    out = pl.pallas_call(
        _final_body,
        grid=(N // NB3,),
        in_specs=[
            pl.BlockSpec((6, NB3, 128), lambda i: (0, i, 0)),
            full((1, H)), full((1, H)),
            full((H, 2 * H)), full((1, 2 * H)),
            full((2 * H, H)), full((2 * H, H)), full((2 * H, H)), full((3, H)),
            full((H, H)), full((H, H)), full((H, H)),
        ],
        out_specs=pl.BlockSpec((9, NB3, H), lambda i: (0, i, 0)),
        out_shape=jax.ShapeDtypeStruct((9, N, H), jnp.float32),
    )(w, ln_g[None, :], ln_b[None, :], ls0_W.T, ls0_b[None, :],
      ls1_W[0::3, :].T, ls1_W[1::3, :].T, ls1_W[2::3, :].T, ls1bias,
      lt0_W.T, lt1_W.T, lt2_W.T)
    return out


@jax.jit
def kernel(z, t, edge_index, edge_weight, edge_vec_norm, edge_attr, node_attr,
           emb_table, mix1_W, mix1_b, mix2_W, mix2_b, emb2_W, emb2_b,
           dp1_W, dp1_b, dp2_W, dp2_b, dp3_W, dp3_b,
           lt0_W, lt1_W, lt2_W, ls0_W, ls0_b, ls1_W, ls1_b, ln_g, ln_b):
    pq = _node_prep(z, t, emb_table, mix1_W, mix1_b, mix2_W, mix2_b,
                    emb2_W, emb2_b)
    a1, a2, a3, g = _edge_coef(edge_weight, edge_vec_norm, edge_attr,
                               dp1_W, dp1_b, dp2_W, dp2_b, dp3_W, dp3_b)
    zr = jnp.zeros((ROWS_PT, 128), jnp.float32)
    ei = edge_index.astype(jnp.int32)
    w = _sc_edge(ei[0], ei[1], pq, a1, a2, a3, g, zr)
    planes = _node_final(w, ln_g, ln_b, ls0_W, ls0_b, ls1_W, ls1_b,
                         lt0_W, lt1_W, lt2_W)
    return planes.transpose(1, 2, 0).reshape(N, H, 3, 3)


# speedup vs baseline: 27.5423x; 1.8077x over previous
import functools

import jax
import jax.numpy as jnp
from jax import lax
from jax.experimental import pallas as pl
from jax.experimental.pallas import tpu as pltpu, tpu_sc as plsc

N = 10000
E = 160000
H = 64
RBF = 32
CUTOFF_UPPER = 5.0


def _silu(x):
    return x * jax.nn.sigmoid(x)


def _node_prep_body(z_ref, t_ref, m1p_ref, m1b_ref,
                    m2_ref, m2b_ref, e2a_ref, e2b_ref, e2bias_ref,
                    p_ref):
    z = z_ref[...]
    onehot = (z == lax.broadcasted_iota(jnp.int32, (1, 128), 1)).astype(jnp.float32)
    Z = jnp.dot(onehot, m1p_ref[...][:128, :], preferred_element_type=jnp.float32,
                precision=lax.Precision.HIGHEST)
    t = t_ref[...]
    zc = jnp.concatenate([Z, t, jnp.zeros((Z.shape[0], 63), jnp.float32)], axis=1)
    pre = jnp.dot(zc, m1p_ref[...][128:, :], preferred_element_type=jnp.float32) \
        + m1b_ref[...]
    h1 = _silu(pre)
    zm = jnp.dot(h1, m2_ref[...], preferred_element_type=jnp.float32) + m2b_ref[...]
    p_ref[:, :H] = jnp.dot(zm, e2a_ref[...], preferred_element_type=jnp.float32)
    p_ref[:, H:] = (jnp.dot(zm, e2b_ref[...], preferred_element_type=jnp.float32)
                    + e2bias_ref[...])


def _node_prep(z, t, emb_table, mix1_W, mix1_b, mix2_W, mix2_b, emb2_W, emb2_b):
    emb_pad = jnp.concatenate([emb_table,
                               jnp.zeros((128 - emb_table.shape[0], H),
                                         jnp.float32)], axis=0)
    m1pad = jnp.concatenate([mix1_W.T, jnp.zeros((63, H), jnp.float32)], axis=0)
    m1p = jnp.concatenate([emb_pad, m1pad], axis=0)
    return pl.pallas_call(
        _node_prep_body,
        out_shape=jax.ShapeDtypeStruct((N, 2 * H), jnp.float32),
    )(z.reshape(N, 1).astype(jnp.int32), t, m1p,
      mix1_b[None, :], mix2_W.T, mix2_b[None, :],
      emb2_W[:, :H].T, emb2_W[:, H:].T, emb2_b[None, :])


EB = 2000
NBLK = E // EB


def _coef_body(ew_ref, evn_ref, ea_ref, dp1_ref, dp2_ref, dp3_ref, db_ref,
               a_ref, a2_ref, a3_ref, g_ref):
    ew = ew_ref[...]
    c = 0.5 * (jnp.cos(ew * (jnp.pi / CUTOFF_UPPER)) + 1.0)
    c = c * (ew < CUTOFF_UPPER).astype(jnp.float32)
    ea = ea_ref[...]
    a1 = (jnp.dot(ea, dp1_ref[...], preferred_element_type=jnp.float32)
          + db_ref[0, :][None, :]) * c
    a2 = (jnp.dot(ea, dp2_ref[...], preferred_element_type=jnp.float32)
          + db_ref[1, :][None, :]) * c
    a3 = (jnp.dot(ea, dp3_ref[...], preferred_element_type=jnp.float32)
          + db_ref[2, :][None, :]) * c
    a_ref[...] = a1
    a2_ref[...] = a2
    a3_ref[...] = a3
    v = evn_ref[...]
    vx, vy, vz = v[:, 0:1], v[:, 1:2], v[:, 2:3]
    m = (vx * vx + vy * vy + vz * vz) * (1.0 / 3.0)
    g_ref[...] = jnp.concatenate(
        [vx, vy, vz, vx * vx - m, vy * vy - m, vz * vz - m,
         vx * vy, vx * vz, vy * vz,
         jnp.zeros((vx.shape[0], 7), jnp.float32)], axis=1)


def _edge_coef(edge_weight, edge_vec_norm, edge_attr,
               dp1_W, dp1_b, dp2_W, dp2_b, dp3_W, dp3_b):
    db = jnp.stack([dp1_b, dp2_b, dp3_b], axis=0)
    full = lambda shape: pl.BlockSpec(shape, lambda i: (0,) * len(shape))
    return pl.pallas_call(
        _coef_body,
        grid=(NBLK,),
        in_specs=[
            pl.BlockSpec((EB, 1), lambda i: (i, 0)),
            pl.BlockSpec((EB, 3), lambda i: (i, 0)),
            pl.BlockSpec((EB, RBF), lambda i: (i, 0)),
            full((RBF, H)), full((RBF, H)), full((RBF, H)), full((3, H)),
        ],
        out_specs=(pl.BlockSpec((EB, H), lambda i: (i, 0)),
                   pl.BlockSpec((EB, H), lambda i: (i, 0)),
                   pl.BlockSpec((EB, H), lambda i: (i, 0)),
                   pl.BlockSpec((EB, 16), lambda i: (i, 0))),
        out_shape=(jax.ShapeDtypeStruct((E, H), jnp.float32),
                   jax.ShapeDtypeStruct((E, H), jnp.float32),
                   jax.ShapeDtypeStruct((E, H), jnp.float32),
                   jax.ShapeDtypeStruct((E, 16), jnp.float32)),
    )(edge_weight.reshape(E, 1), edge_vec_norm, edge_attr,
      dp1_W.T, dp2_W.T, dp3_W.T, db)


C = 40
EPT = E // 16
NCHUNK = EPT // C
SPLIT0 = 124
NPAD = 10240
ROWS_PT = NPAD // 16

BLOCK_AK = [0, 1, 1, 1, 2, 2, 2, 2, 2, 2]
SLOT_BLOCKS = [(0, 1), (2, 3), (4, 5), (6, 7), (8, 9)]

_mesh = plsc.VectorSubcoreMesh(core_axis_name="c", subcore_axis_name="s")


@functools.partial(
    pl.kernel, mesh=_mesh,
    out_type=jax.ShapeDtypeStruct((6, NPAD, 128), jnp.float32),
    scratch_types=[
        pltpu.VMEM((C,), jnp.int32),
        pltpu.VMEM((C,), jnp.int32),
        pltpu.VMEM((C,), jnp.int32),
        pltpu.VMEM((C,), jnp.int32),
        pltpu.VMEM((C, 2 * H), jnp.float32),
        pltpu.VMEM((C, 2 * H), jnp.float32),
        pltpu.VMEM((C, H), jnp.float32),
        pltpu.VMEM((C, H), jnp.float32),
        pltpu.VMEM((C, 16), jnp.float32),
        pltpu.VMEM((C, 2 * H), jnp.float32),
        pltpu.VMEM_SHARED((NPAD, 128), jnp.float32),
        pltpu.SemaphoreType.DMA,
        pltpu.SemaphoreType.DMA,
    ],
)
def _sc_edge(srcs_hbm, dsts_hbm, pq_hbm, a1_hbm, a2_hbm, a3_hbm, g_hbm,
             zr_hbm, w_hbm,
             src0_v, src1_v, dst0_v, dst1_v, p_v, q_v, a0_v, a1_v, g_v, pay_v,
             acc, semA, semB):
    core = lax.axis_index("c")
    sid = lax.axis_index("s")

    ak_hbm = [a1_hbm, a2_hbm, a3_hbm]
    src_b = [src0_v, src1_v]
    dst_b = [dst0_v, dst1_v]

    def _fire_idx(k, b):
        base = sid * EPT + k * C
        pltpu.async_copy(srcs_hbm.at[pl.ds(base, C)], src_b[b], semB)
        pltpu.async_copy(dsts_hbm.at[pl.ds(base, C)], dst_b[b], semB)

    def _wait_idx(b):
        pltpu.make_async_copy(srcs_hbm.at[pl.ds(0, C)], src_b[b], semB).wait()
        pltpu.make_async_copy(dsts_hbm.at[pl.ds(0, C)], dst_b[b], semB).wait()

    def run_chunks(k_lo, k_hi, blocks):
        k0, k1 = BLOCK_AK[blocks[0]], BLOCK_AK[blocks[1]]
        stages = [a0_v, a1_v if k1 != k0 else a0_v]
        _fire_idx(k_lo, 0)

        def chunk_b(k, b):
            base = sid * EPT + k * C
            _wait_idx(b)
            hs = [pltpu.async_copy(pq_hbm.at[src_b[b]], p_v, semA),
                  pltpu.async_copy(pq_hbm.at[dst_b[b]], q_v, semA),
                  pltpu.async_copy(ak_hbm[k0].at[pl.ds(base, C)], a0_v, semA)]
            if k1 != k0:
                hs.append(pltpu.async_copy(ak_hbm[k1].at[pl.ds(base, C)], a1_v, semA))
            hs.append(pltpu.async_copy(g_hbm.at[pl.ds(base, C)], g_v, semA))

            @pl.when(k + 1 < k_hi)
            def _prefetch():
                _fire_idx(k + 1, 1 - b)

            for h in hs:
                h.wait()

            def edge(e, _):
                zs = [p_v[e, pl.ds(16 * hb, 16)] + q_v[e, pl.ds(64 + 16 * hb, 16)]
                      for hb in range(4)]
                for b, j in enumerate(blocks):
                    a_v = stages[b]
                    if j == 0:
                        for hb in range(4):
                            d = a_v[e, pl.ds(16 * hb, 16)] * zs[hb]
                            pay_v[e, pl.ds(64 * b + 16 * hb, 16)] = d
                    else:
                        g = g_v[e, :][j - 1]
                        for hb in range(4):
                            d = a_v[e, pl.ds(16 * hb, 16)] * zs[hb]
                            pay_v[e, pl.ds(64 * b + 16 * hb, 16)] = d * g
                return 0
            lax.fori_loop(0, C, edge, 0, unroll=False)
            pltpu.sync_copy(pay_v, acc.at[src_b[b]], add=True)

        def pair(k2, _):
            for b in range(2):
                chunk_b(k_lo + 2 * k2 + b, b)
            return 0
        lax.fori_loop(0, (k_hi - k_lo) // 2, pair, 0, unroll=False)

    for p in range(3):
        pltpu.sync_copy(zr_hbm, acc.at[pl.ds(sid * ROWS_PT, ROWS_PT)])
        plsc.subcore_barrier()
        if p < 2:
            bl0, bl1 = SLOT_BLOCKS[2 * p], SLOT_BLOCKS[2 * p + 1]

            @pl.when(core == 0)
            def _c0():
                run_chunks(0, NCHUNK, bl0)

            @pl.when(core == 1)
            def _c1():
                run_chunks(0, NCHUNK, bl1)
        else:
            bl = SLOT_BLOCKS[4]

            @pl.when(core == 0)
            def _c0():
                run_chunks(0, SPLIT0, bl)

            @pl.when(core == 1)
            def _c1():
                run_chunks(SPLIT0, NCHUNK, bl)

        plsc.subcore_barrier()
        pltpu.sync_copy(acc.at[pl.ds(sid * ROWS_PT, ROWS_PT)],
                        w_hbm.at[2 * p + core, pl.ds(sid * ROWS_PT, ROWS_PT)])
        plsc.subcore_barrier()


NB3 = 2000


def _final_body(w_ref, lng_ref, lnb_ref, ls0_ref, ls0b_ref,
                ls1a_ref, ls1b_ref, ls1c_ref, ls1bias_ref,
                lt0_ref, lt1_ref, lt2_ref, o_ref):
    w = w_ref[...]
    s1 = w[0, :, 0:H]
    w2 = [w[0, :, H:2 * H], w[1, :, 0:H], w[1, :, H:2 * H]]
    w3 = [w[2, :, 0:H], w[2, :, H:2 * H],
          w[3, :, 0:H], w[3, :, H:2 * H],
          w[4, :, 0:H] + w[5, :, 0:H], w[4, :, H:2 * H] + w[5, :, H:2 * H]]
    nrm = (3.0 * s1 * s1
           + 2.0 * (w2[0] * w2[0] + w2[1] * w2[1] + w2[2] * w2[2])
           + w3[0] * w3[0] + w3[1] * w3[1] + w3[2] * w3[2]
           + 2.0 * (w3[3] * w3[3] + w3[4] * w3[4] + w3[5] * w3[5]))
    nrm = jnp.maximum(nrm, 0.01)
    mu = jnp.mean(nrm, axis=-1, keepdims=True)
    var = jnp.mean((nrm - mu) ** 2, axis=-1, keepdims=True)
    nrm = (nrm - mu) / jnp.sqrt(var + 1e-5) * lng_ref[...] + lnb_ref[...]
    f = _silu(jnp.dot(nrm, ls0_ref[...], preferred_element_type=jnp.float32)
              + ls0b_ref[...])
    n0 = _silu(jnp.dot(f, ls1a_ref[...], preferred_element_type=jnp.float32)
               + ls1bias_ref[0, :][None, :])
    n1 = _silu(jnp.dot(f, ls1b_ref[...], preferred_element_type=jnp.float32)
               + ls1bias_ref[1, :][None, :])
    n2 = _silu(jnp.dot(f, ls1c_ref[...], preferred_element_type=jnp.float32)
               + ls1bias_ref[2, :][None, :])
    ip = jnp.dot(s1, lt0_ref[...], preferred_element_type=jnp.float32) * n0
    w2p = [jnp.dot(w2[i], lt1_ref[...], preferred_element_type=jnp.float32) * n1
           for i in range(3)]
    w3p = [jnp.dot(w3[i], lt2_ref[...], preferred_element_type=jnp.float32) * n2
           for i in range(6)]
    o_ref[0] = ip + w3p[0]
    o_ref[1] = -w2p[2] + w3p[3]
    o_ref[2] = w2p[1] + w3p[4]
    o_ref[3] = w2p[2] + w3p[3]
    o_ref[4] = ip + w3p[1]
    o_ref[5] = -w2p[0] + w3p[5]
    o_ref[6] = -w2p[1] + w3p[4]
    o_ref[7] = w2p[0] + w3p[5]
    o_ref[8] = ip + w3p[2]


def _node_final(w, ln_g, ln_b, ls0_W, ls0_b, ls1_W, ls1_b, lt0_W, lt1_W, lt2_W):
    ls1bias = jnp.stack([ls1_b[0::3], ls1_b[1::3], ls1_b[2::3]], axis=0)
    full = lambda shape: pl.BlockSpec(shape, lambda i: (0,) * len(shape))
    out = pl.pallas_call(
        _final_body,
        grid=(N // NB3,),
        in_specs=[
            pl.BlockSpec((6, NB3, 128), lambda i: (0, i, 0)),
            full((1, H)), full((1, H)),
            full((H, 2 * H)), full((1, 2 * H)),
            full((2 * H, H)), full((2 * H, H)), full((2 * H, H)), full((3, H)),
            full((H, H)), full((H, H)), full((H, H)),
        ],
        out_specs=pl.BlockSpec((9, NB3, H), lambda i: (0, i, 0)),
        out_shape=jax.ShapeDtypeStruct((9, N, H), jnp.float32),
    )(w, ln_g[None, :], ln_b[None, :], ls0_W.T, ls0_b[None, :],
      ls1_W[0::3, :].T, ls1_W[1::3, :].T, ls1_W[2::3, :].T, ls1bias,
      lt0_W.T, lt1_W.T, lt2_W.T)
    return out


@jax.jit
def kernel(z, t, edge_index, edge_weight, edge_vec_norm, edge_attr, node_attr,
           emb_table, mix1_W, mix1_b, mix2_W, mix2_b, emb2_W, emb2_b,
           dp1_W, dp1_b, dp2_W, dp2_b, dp3_W, dp3_b,
           lt0_W, lt1_W, lt2_W, ls0_W, ls0_b, ls1_W, ls1_b, ln_g, ln_b):
    pq = _node_prep(z, t, emb_table, mix1_W, mix1_b, mix2_W, mix2_b,
                    emb2_W, emb2_b)
    a1, a2, a3, g = _edge_coef(edge_weight, edge_vec_norm, edge_attr,
                               dp1_W, dp1_b, dp2_W, dp2_b, dp3_W, dp3_b)
    zr = jnp.zeros((ROWS_PT, 128), jnp.float32)
    ei = edge_index.astype(jnp.int32)
    w = _sc_edge(ei[0], ei[1], pq, a1, a2, a3, g, zr)
    planes = _node_final(w, ln_g, ln_b, ls0_W, ls0_b, ls1_W, ls1_b,
                         lt0_W, lt1_W, lt2_W)
    return planes.transpose(1, 2, 0).reshape(N, H, 3, 3)


# speedup vs baseline: 29.4428x; 1.0690x over previous
import functools

import jax
import jax.numpy as jnp
from jax import lax
from jax.experimental import pallas as pl
from jax.experimental.pallas import tpu as pltpu, tpu_sc as plsc

N = 10000
E = 160000
H = 64
RBF = 32
CUTOFF_UPPER = 5.0


def _silu(x):
    return x * jax.nn.sigmoid(x)


def _node_prep_body(z_ref, t_ref, m1p_ref, m1b_ref,
                    m2_ref, m2b_ref, e2a_ref, e2b_ref, e2bias_ref,
                    p_ref):
    z = z_ref[...]
    onehot = (z == lax.broadcasted_iota(jnp.int32, (1, 128), 1)).astype(jnp.float32)
    Z = jnp.dot(onehot, m1p_ref[...][:128, :], preferred_element_type=jnp.float32,
                precision=lax.Precision.HIGHEST)
    t = t_ref[...]
    zc = jnp.concatenate([Z, t, jnp.zeros((Z.shape[0], 63), jnp.float32)], axis=1)
    pre = jnp.dot(zc, m1p_ref[...][128:, :], preferred_element_type=jnp.float32) \
        + m1b_ref[...]
    h1 = _silu(pre)
    zm = jnp.dot(h1, m2_ref[...], preferred_element_type=jnp.float32) + m2b_ref[...]
    p_ref[:, :H] = jnp.dot(zm, e2a_ref[...], preferred_element_type=jnp.float32)
    p_ref[:, H:] = (jnp.dot(zm, e2b_ref[...], preferred_element_type=jnp.float32)
                    + e2bias_ref[...])


def _node_prep(z, t, emb_table, mix1_W, mix1_b, mix2_W, mix2_b, emb2_W, emb2_b):
    emb_pad = jnp.concatenate([emb_table,
                               jnp.zeros((128 - emb_table.shape[0], H),
                                         jnp.float32)], axis=0)
    m1pad = jnp.concatenate([mix1_W.T, jnp.zeros((63, H), jnp.float32)], axis=0)
    m1p = jnp.concatenate([emb_pad, m1pad], axis=0)
    return pl.pallas_call(
        _node_prep_body,
        out_shape=jax.ShapeDtypeStruct((N, 2 * H), jnp.float32),
    )(z.reshape(N, 1).astype(jnp.int32), t, m1p,
      mix1_b[None, :], mix2_W.T, mix2_b[None, :],
      emb2_W[:, :H].T, emb2_W[:, H:].T, emb2_b[None, :])


EB = 2000
NBLK = E // EB


def _coef_body(ew_ref, evn_ref, ea_ref, dp1_ref, dp2_ref, dp3_ref, db_ref,
               a_ref, a2_ref, a3_ref, g_ref):
    ew = ew_ref[...]
    c = 0.5 * (jnp.cos(ew * (jnp.pi / CUTOFF_UPPER)) + 1.0)
    c = c * (ew < CUTOFF_UPPER).astype(jnp.float32)
    ea = ea_ref[...]
    a1 = (jnp.dot(ea, dp1_ref[...], preferred_element_type=jnp.float32)
          + db_ref[0, :][None, :]) * c
    a2 = (jnp.dot(ea, dp2_ref[...], preferred_element_type=jnp.float32)
          + db_ref[1, :][None, :]) * c
    a3 = (jnp.dot(ea, dp3_ref[...], preferred_element_type=jnp.float32)
          + db_ref[2, :][None, :]) * c
    a_ref[...] = a1
    a2_ref[...] = a2
    a3_ref[...] = a3
    v = evn_ref[...]
    vx, vy, vz = v[:, 0:1], v[:, 1:2], v[:, 2:3]
    m = (vx * vx + vy * vy + vz * vz) * (1.0 / 3.0)
    g_ref[...] = jnp.concatenate(
        [vx, vy, vz, vx * vx - m, vy * vy - m, vz * vz - m,
         vx * vy, vx * vz, vy * vz,
         jnp.zeros((vx.shape[0], 7), jnp.float32)], axis=1)


def _edge_coef(edge_weight, edge_vec_norm, edge_attr,
               dp1_W, dp1_b, dp2_W, dp2_b, dp3_W, dp3_b):
    db = jnp.stack([dp1_b, dp2_b, dp3_b], axis=0)
    full = lambda shape: pl.BlockSpec(shape, lambda i: (0,) * len(shape))
    return pl.pallas_call(
        _coef_body,
        grid=(NBLK,),
        in_specs=[
            pl.BlockSpec((EB, 1), lambda i: (i, 0)),
            pl.BlockSpec((EB, 3), lambda i: (i, 0)),
            pl.BlockSpec((EB, RBF), lambda i: (i, 0)),
            full((RBF, H)), full((RBF, H)), full((RBF, H)), full((3, H)),
        ],
        out_specs=(pl.BlockSpec((EB, H), lambda i: (i, 0)),
                   pl.BlockSpec((EB, H), lambda i: (i, 0)),
                   pl.BlockSpec((EB, H), lambda i: (i, 0)),
                   pl.BlockSpec((EB, 16), lambda i: (i, 0))),
        out_shape=(jax.ShapeDtypeStruct((E, H), jnp.float32),
                   jax.ShapeDtypeStruct((E, H), jnp.float32),
                   jax.ShapeDtypeStruct((E, H), jnp.float32),
                   jax.ShapeDtypeStruct((E, 16), jnp.float32)),
    )(edge_weight.reshape(E, 1), edge_vec_norm, edge_attr,
      dp1_W.T, dp2_W.T, dp3_W.T, db)


C = 40
EPT = E // 16
NCHUNK = EPT // C
SPLIT0 = 124
NPAD = 10240
ROWS_PT = NPAD // 16

BLOCK_AK = [0, 1, 1, 1, 2, 2, 2, 2, 2, 2]
SLOT_BLOCKS = [(0, 1), (2, 3), (4, 5), (6, 7), (8, 9)]

_mesh = plsc.VectorSubcoreMesh(core_axis_name="c", subcore_axis_name="s")


@functools.partial(
    pl.kernel, mesh=_mesh,
    out_type=jax.ShapeDtypeStruct((6, NPAD, 128), jnp.float32),
    scratch_types=[
        pltpu.VMEM((C,), jnp.int32),
        pltpu.VMEM((C,), jnp.int32),
        pltpu.VMEM((C,), jnp.int32),
        pltpu.VMEM((C,), jnp.int32),
        pltpu.VMEM((C, 2 * H), jnp.float32),
        pltpu.VMEM((C, 2 * H), jnp.float32),
        pltpu.VMEM((C, H), jnp.float32),
        pltpu.VMEM((C, H), jnp.float32),
        pltpu.VMEM((C, 16), jnp.float32),
        pltpu.VMEM((C, 2 * H), jnp.float32),
        pltpu.VMEM((C, 2 * H), jnp.float32),
        pltpu.VMEM_SHARED((NPAD, 128), jnp.float32),
        pltpu.SemaphoreType.DMA,
        pltpu.SemaphoreType.DMA,
        pltpu.SemaphoreType.DMA,
    ],
)
def _sc_edge(srcs_hbm, dsts_hbm, pq_hbm, a1_hbm, a2_hbm, a3_hbm, g_hbm,
             zr_hbm, w_hbm,
             src0_v, src1_v, dst0_v, dst1_v, p_v, q_v, a0_v, a1_v, g_v, pay0_v,
             pay1_v, acc, semA, semB, semC):
    core = lax.axis_index("c")
    sid = lax.axis_index("s")

    ak_hbm = [a1_hbm, a2_hbm, a3_hbm]
    src_b = [src0_v, src1_v]
    dst_b = [dst0_v, dst1_v]
    pay_b = [pay0_v, pay1_v]

    def _fire_idx(k, b):
        base = sid * EPT + k * C
        pltpu.async_copy(srcs_hbm.at[pl.ds(base, C)], src_b[b], semB)
        pltpu.async_copy(dsts_hbm.at[pl.ds(base, C)], dst_b[b], semB)

    def _wait_idx(b):
        pltpu.make_async_copy(srcs_hbm.at[pl.ds(0, C)], src_b[b], semB).wait()
        pltpu.make_async_copy(dsts_hbm.at[pl.ds(0, C)], dst_b[b], semB).wait()

    def run_chunks(k_lo, k_hi, blocks):
        k0, k1 = BLOCK_AK[blocks[0]], BLOCK_AK[blocks[1]]
        stages = [a0_v, a1_v if k1 != k0 else a0_v]
        _fire_idx(k_lo, 0)

        def chunk_b(k, b):
            base = sid * EPT + k * C
            pay_v = pay_b[b]
            _wait_idx(b)
            hs = [pltpu.async_copy(pq_hbm.at[src_b[b]], p_v, semA),
                  pltpu.async_copy(pq_hbm.at[dst_b[b]], q_v, semA),
                  pltpu.async_copy(ak_hbm[k0].at[pl.ds(base, C)], a0_v, semA)]
            if k1 != k0:
                hs.append(pltpu.async_copy(ak_hbm[k1].at[pl.ds(base, C)], a1_v, semA))
            hs.append(pltpu.async_copy(g_hbm.at[pl.ds(base, C)], g_v, semA))

            @pl.when(k > k_lo)
            def _drain_prev():
                pltpu.make_async_copy(pay_b[1 - b], acc.at[src_b[1 - b]],
                                      semC).wait()

            @pl.when(k + 1 < k_hi)
            def _prefetch():
                _fire_idx(k + 1, 1 - b)

            for h in hs:
                h.wait()

            def edge(e, _):
                zs = [p_v[e, pl.ds(16 * hb, 16)] + q_v[e, pl.ds(64 + 16 * hb, 16)]
                      for hb in range(4)]
                for b, j in enumerate(blocks):
                    a_v = stages[b]
                    if j == 0:
                        for hb in range(4):
                            d = a_v[e, pl.ds(16 * hb, 16)] * zs[hb]
                            pay_v[e, pl.ds(64 * b + 16 * hb, 16)] = d
                    else:
                        g = g_v[e, :][j - 1]
                        for hb in range(4):
                            d = a_v[e, pl.ds(16 * hb, 16)] * zs[hb]
                            pay_v[e, pl.ds(64 * b + 16 * hb, 16)] = d * g
                return 0
            lax.fori_loop(0, C, edge, 0, unroll=False)
            pltpu.async_copy(pay_v, acc.at[src_b[b]], semC, add=True)

        def pair(k2, _):
            for b in range(2):
                chunk_b(k_lo + 2 * k2 + b, b)
            return 0
        lax.fori_loop(0, (k_hi - k_lo) // 2, pair, 0, unroll=False)
        pltpu.make_async_copy(pay_b[1], acc.at[src_b[1]], semC).wait()

    for p in range(3):
        pltpu.sync_copy(zr_hbm, acc.at[pl.ds(sid * ROWS_PT, ROWS_PT)])
        plsc.subcore_barrier()
        if p < 2:
            bl0, bl1 = SLOT_BLOCKS[2 * p], SLOT_BLOCKS[2 * p + 1]

            @pl.when(core == 0)
            def _c0():
                run_chunks(0, NCHUNK, bl0)

            @pl.when(core == 1)
            def _c1():
                run_chunks(0, NCHUNK, bl1)
        else:
            bl = SLOT_BLOCKS[4]

            @pl.when(core == 0)
            def _c0():
                run_chunks(0, SPLIT0, bl)

            @pl.when(core == 1)
            def _c1():
                run_chunks(SPLIT0, NCHUNK, bl)

        plsc.subcore_barrier()
        pltpu.sync_copy(acc.at[pl.ds(sid * ROWS_PT, ROWS_PT)],
                        w_hbm.at[2 * p + core, pl.ds(sid * ROWS_PT, ROWS_PT)])
        plsc.subcore_barrier()


NB3 = 2000


def _final_body(w_ref, lng_ref, lnb_ref, ls0_ref, ls0b_ref,
                ls1a_ref, ls1b_ref, ls1c_ref, ls1bias_ref,
                lt0_ref, lt1_ref, lt2_ref, o_ref):
    w = w_ref[...]
    s1 = w[0, :, 0:H]
    w2 = [w[0, :, H:2 * H], w[1, :, 0:H], w[1, :, H:2 * H]]
    w3 = [w[2, :, 0:H], w[2, :, H:2 * H],
          w[3, :, 0:H], w[3, :, H:2 * H],
          w[4, :, 0:H] + w[5, :, 0:H], w[4, :, H:2 * H] + w[5, :, H:2 * H]]
    nrm = (3.0 * s1 * s1
           + 2.0 * (w2[0] * w2[0] + w2[1] * w2[1] + w2[2] * w2[2])
           + w3[0] * w3[0] + w3[1] * w3[1] + w3[2] * w3[2]
           + 2.0 * (w3[3] * w3[3] + w3[4] * w3[4] + w3[5] * w3[5]))
    nrm = jnp.maximum(nrm, 0.01)
    mu = jnp.mean(nrm, axis=-1, keepdims=True)
    var = jnp.mean((nrm - mu) ** 2, axis=-1, keepdims=True)
    nrm = (nrm - mu) / jnp.sqrt(var + 1e-5) * lng_ref[...] + lnb_ref[...]
    f = _silu(jnp.dot(nrm, ls0_ref[...], preferred_element_type=jnp.float32)
              + ls0b_ref[...])
    n0 = _silu(jnp.dot(f, ls1a_ref[...], preferred_element_type=jnp.float32)
               + ls1bias_ref[0, :][None, :])
    n1 = _silu(jnp.dot(f, ls1b_ref[...], preferred_element_type=jnp.float32)
               + ls1bias_ref[1, :][None, :])
    n2 = _silu(jnp.dot(f, ls1c_ref[...], preferred_element_type=jnp.float32)
               + ls1bias_ref[2, :][None, :])
    ip = jnp.dot(s1, lt0_ref[...], preferred_element_type=jnp.float32) * n0
    w2p = [jnp.dot(w2[i], lt1_ref[...], preferred_element_type=jnp.float32) * n1
           for i in range(3)]
    w3p = [jnp.dot(w3[i], lt2_ref[...], preferred_element_type=jnp.float32) * n2
           for i in range(6)]
    o_ref[0] = ip + w3p[0]
    o_ref[1] = -w2p[2] + w3p[3]
    o_ref[2] = w2p[1] + w3p[4]
    o_ref[3] = w2p[2] + w3p[3]
    o_ref[4] = ip + w3p[1]
    o_ref[5] = -w2p[0] + w3p[5]
    o_ref[6] = -w2p[1] + w3p[4]
    o_ref[7] = w2p[0] + w3p[5]
    o_ref[8] = ip + w3p[2]


def _node_final(w, ln_g, ln_b, ls0_W, ls0_b, ls1_W, ls1_b, lt0_W, lt1_W, lt2_W):
    ls1bias = jnp.stack([ls1_b[0::3], ls1_b[1::3], ls1_b[2::3]], axis=0)
    full = lambda shape: pl.BlockSpec(shape, lambda i: (0,) * len(shape))
    out = pl.pallas_call(
        _final_body,
        grid=(N // NB3,),
        in_specs=[
            pl.BlockSpec((6, NB3, 128), lambda i: (0, i, 0)),
            full((1, H)), full((1, H)),
            full((H, 2 * H)), full((1, 2 * H)),
            full((2 * H, H)), full((2 * H, H)), full((2 * H, H)), full((3, H)),
            full((H, H)), full((H, H)), full((H, H)),
        ],
        out_specs=pl.BlockSpec((9, NB3, H), lambda i: (0, i, 0)),
        out_shape=jax.ShapeDtypeStruct((9, N, H), jnp.float32),
    )(w, ln_g[None, :], ln_b[None, :], ls0_W.T, ls0_b[None, :],
      ls1_W[0::3, :].T, ls1_W[1::3, :].T, ls1_W[2::3, :].T, ls1bias,
      lt0_W.T, lt1_W.T, lt2_W.T)
    return out


@jax.jit
def kernel(z, t, edge_index, edge_weight, edge_vec_norm, edge_attr, node_attr,
           emb_table, mix1_W, mix1_b, mix2_W, mix2_b, emb2_W, emb2_b,
           dp1_W, dp1_b, dp2_W, dp2_b, dp3_W, dp3_b,
           lt0_W, lt1_W, lt2_W, ls0_W, ls0_b, ls1_W, ls1_b, ln_g, ln_b):
    pq = _node_prep(z, t, emb_table, mix1_W, mix1_b, mix2_W, mix2_b,
                    emb2_W, emb2_b)
    a1, a2, a3, g = _edge_coef(edge_weight, edge_vec_norm, edge_attr,
                               dp1_W, dp1_b, dp2_W, dp2_b, dp3_W, dp3_b)
    zr = jnp.zeros((ROWS_PT, 128), jnp.float32)
    ei = edge_index.astype(jnp.int32)
    w = _sc_edge(ei[0], ei[1], pq, a1, a2, a3, g, zr)
    planes = _node_final(w, ln_g, ln_b, ls0_W, ls0_b, ls1_W, ls1_b,
                         lt0_W, lt1_W, lt2_W)
    return planes.transpose(1, 2, 0).reshape(N, H, 3, 3)


# speedup vs baseline: 37.0168x; 1.2572x over previous
import functools

import jax
import jax.numpy as jnp
from jax import lax
from jax.experimental import pallas as pl
from jax.experimental.pallas import tpu as pltpu, tpu_sc as plsc

N = 10000
E = 160000
H = 64
RBF = 32
CUTOFF_UPPER = 5.0


def _silu(x):
    return x * jax.nn.sigmoid(x)


def _node_prep_body(z_ref, t_ref, m1p_ref, m1b_ref,
                    m2_ref, m2b_ref, e2a_ref, e2b_ref, e2bias_ref,
                    p_ref):
    z = z_ref[...]
    onehot = (z == lax.broadcasted_iota(jnp.int32, (1, 128), 1)).astype(jnp.float32)
    Z = jnp.dot(onehot, m1p_ref[...][:128, :], preferred_element_type=jnp.float32,
                precision=lax.Precision.HIGHEST)
    t = t_ref[...]
    zc = jnp.concatenate([Z, t, jnp.zeros((Z.shape[0], 63), jnp.float32)], axis=1)
    pre = jnp.dot(zc, m1p_ref[...][128:, :], preferred_element_type=jnp.float32) \
        + m1b_ref[...]
    h1 = _silu(pre)
    zm = jnp.dot(h1, m2_ref[...], preferred_element_type=jnp.float32) + m2b_ref[...]
    p_ref[:, :H] = jnp.dot(zm, e2a_ref[...], preferred_element_type=jnp.float32)
    p_ref[:, H:] = (jnp.dot(zm, e2b_ref[...], preferred_element_type=jnp.float32)
                    + e2bias_ref[...])


def _node_prep(z, t, emb_table, mix1_W, mix1_b, mix2_W, mix2_b, emb2_W, emb2_b):
    emb_pad = jnp.concatenate([emb_table,
                               jnp.zeros((128 - emb_table.shape[0], H),
                                         jnp.float32)], axis=0)
    m1pad = jnp.concatenate([mix1_W.T, jnp.zeros((63, H), jnp.float32)], axis=0)
    m1p = jnp.concatenate([emb_pad, m1pad], axis=0)
    return pl.pallas_call(
        _node_prep_body,
        out_shape=jax.ShapeDtypeStruct((N, 2 * H), jnp.float32),
    )(z.reshape(N, 1).astype(jnp.int32), t, m1p,
      mix1_b[None, :], mix2_W.T, mix2_b[None, :],
      emb2_W[:, :H].T, emb2_W[:, H:].T, emb2_b[None, :])


EB = 2000
NBLK = E // EB


def _coef_body(ea_ref, dp1_ref, dp2_ref, dp3_ref, db_ref,
               a_ref, a2_ref, a3_ref):
    ea = ea_ref[...]
    a_ref[...] = (jnp.dot(ea, dp1_ref[...], preferred_element_type=jnp.float32)
                  + db_ref[0, :][None, :])
    a2_ref[...] = (jnp.dot(ea, dp2_ref[...], preferred_element_type=jnp.float32)
                   + db_ref[1, :][None, :])
    a3_ref[...] = (jnp.dot(ea, dp3_ref[...], preferred_element_type=jnp.float32)
                   + db_ref[2, :][None, :])


def _edge_coef(edge_attr, dp1_W, dp1_b, dp2_W, dp2_b, dp3_W, dp3_b):
    db = jnp.stack([dp1_b, dp2_b, dp3_b], axis=0)
    full = lambda shape: pl.BlockSpec(shape, lambda i: (0,) * len(shape))
    return pl.pallas_call(
        _coef_body,
        grid=(NBLK,),
        in_specs=[
            pl.BlockSpec((EB, RBF), lambda i: (i, 0)),
            full((RBF, H)), full((RBF, H)), full((RBF, H)), full((3, H)),
        ],
        out_specs=(pl.BlockSpec((EB, H), lambda i: (i, 0)),
                   pl.BlockSpec((EB, H), lambda i: (i, 0)),
                   pl.BlockSpec((EB, H), lambda i: (i, 0))),
        out_shape=(jax.ShapeDtypeStruct((E, H), jnp.float32),
                   jax.ShapeDtypeStruct((E, H), jnp.float32),
                   jax.ShapeDtypeStruct((E, H), jnp.float32)),
    )(edge_attr, dp1_W.T, dp2_W.T, dp3_W.T, db)


EP_R = E // 128


def _geom_body(ew_ref, vx_ref, vy_ref, vz_ref,
               c_ref, u0_ref, u1_ref, u2_ref, u3_ref, u4_ref, u5_ref):
    ew = ew_ref[...]
    c = 0.5 * (jnp.cos(ew * (jnp.pi / CUTOFF_UPPER)) + 1.0)
    c_ref[...] = c * (ew < CUTOFF_UPPER).astype(jnp.float32)
    vx, vy, vz = vx_ref[...], vy_ref[...], vz_ref[...]
    m = (vx * vx + vy * vy + vz * vz) * (1.0 / 3.0)
    u0_ref[...] = vx * vx - m
    u1_ref[...] = vy * vy - m
    u2_ref[...] = vz * vz - m
    u3_ref[...] = vx * vy
    u4_ref[...] = vx * vz
    u5_ref[...] = vy * vz


def _geom(edge_weight, edge_vec_norm):
    ew2 = edge_weight.reshape(EP_R, 128)
    vx = edge_vec_norm[:, 0].reshape(EP_R, 128)
    vy = edge_vec_norm[:, 1].reshape(EP_R, 128)
    vz = edge_vec_norm[:, 2].reshape(EP_R, 128)
    shp = jax.ShapeDtypeStruct((EP_R, 128), jnp.float32)
    c, u0, u1, u2, u3, u4, u5 = pl.pallas_call(
        _geom_body, out_shape=(shp,) * 7)(ew2, vx, vy, vz)
    cols = [edge_vec_norm[:, 0], edge_vec_norm[:, 1], edge_vec_norm[:, 2],
            u0.reshape(E), u1.reshape(E), u2.reshape(E),
            u3.reshape(E), u4.reshape(E), u5.reshape(E), c.reshape(E)]
    zcol = jnp.zeros((E,), jnp.float32)
    return jnp.stack(cols + [zcol] * 6, axis=1)


C = 40
EPT = E // 16
NCHUNK = EPT // C
SPLIT0 = 124
NPAD = 10240
ROWS_PT = NPAD // 16

BLOCK_AK = [0, 1, 1, 1, 2, 2, 2, 2, 2, 2]
SLOT_BLOCKS = [(0, 1), (2, 3), (4, 5), (6, 7), (8, 9)]

_mesh = plsc.VectorSubcoreMesh(core_axis_name="c", subcore_axis_name="s")


@functools.partial(
    pl.kernel, mesh=_mesh,
    out_type=jax.ShapeDtypeStruct((6, NPAD, 128), jnp.float32),
    scratch_types=[
        pltpu.VMEM((C,), jnp.int32),
        pltpu.VMEM((C,), jnp.int32),
        pltpu.VMEM((C,), jnp.int32),
        pltpu.VMEM((C,), jnp.int32),
        pltpu.VMEM((C, 2 * H), jnp.float32),
        pltpu.VMEM((C, 2 * H), jnp.float32),
        pltpu.VMEM((C, H), jnp.float32),
        pltpu.VMEM((C, H), jnp.float32),
        pltpu.VMEM((C, 16), jnp.float32),
        pltpu.VMEM((C, 2 * H), jnp.float32),
        pltpu.VMEM((C, 2 * H), jnp.float32),
        pltpu.VMEM_SHARED((NPAD, 128), jnp.float32),
        pltpu.SemaphoreType.DMA,
        pltpu.SemaphoreType.DMA,
        pltpu.SemaphoreType.DMA,
    ],
)
def _sc_edge(srcs_hbm, dsts_hbm, pq_hbm, a1_hbm, a2_hbm, a3_hbm, g_hbm,
             zr_hbm, w_hbm,
             src0_v, src1_v, dst0_v, dst1_v, p_v, q_v, a0_v, a1_v, g_v, pay0_v,
             pay1_v, acc, semA, semB, semC):
    core = lax.axis_index("c")
    sid = lax.axis_index("s")

    ak_hbm = [a1_hbm, a2_hbm, a3_hbm]
    src_b = [src0_v, src1_v]
    dst_b = [dst0_v, dst1_v]
    pay_b = [pay0_v, pay1_v]

    def _fire_idx(k, b):
        base = sid * EPT + k * C
        pltpu.async_copy(srcs_hbm.at[pl.ds(base, C)], src_b[b], semB)
        pltpu.async_copy(dsts_hbm.at[pl.ds(base, C)], dst_b[b], semB)

    def _wait_idx(b):
        pltpu.make_async_copy(srcs_hbm.at[pl.ds(0, C)], src_b[b], semB).wait()
        pltpu.make_async_copy(dsts_hbm.at[pl.ds(0, C)], dst_b[b], semB).wait()

    def run_chunks(k_lo, k_hi, blocks):
        k0, k1 = BLOCK_AK[blocks[0]], BLOCK_AK[blocks[1]]
        stages = [a0_v, a1_v if k1 != k0 else a0_v]
        _fire_idx(k_lo, 0)

        def chunk_b(k, b):
            base = sid * EPT + k * C
            pay_v = pay_b[b]
            _wait_idx(b)
            hs = [pltpu.async_copy(pq_hbm.at[src_b[b]], p_v, semA),
                  pltpu.async_copy(pq_hbm.at[dst_b[b]], q_v, semA),
                  pltpu.async_copy(ak_hbm[k0].at[pl.ds(base, C)], a0_v, semA)]
            if k1 != k0:
                hs.append(pltpu.async_copy(ak_hbm[k1].at[pl.ds(base, C)], a1_v, semA))
            hs.append(pltpu.async_copy(g_hbm.at[pl.ds(base, C)], g_v, semA))

            @pl.when(k > k_lo)
            def _drain_prev():
                pltpu.make_async_copy(pay_b[1 - b], acc.at[src_b[1 - b]],
                                      semC).wait()

            @pl.when(k + 1 < k_hi)
            def _prefetch():
                _fire_idx(k + 1, 1 - b)

            for h in hs:
                h.wait()

            def edge(e, _):
                grow = g_v[e, :]
                cs = grow[9]
                zs = [(p_v[e, pl.ds(16 * hb, 16)]
                       + q_v[e, pl.ds(64 + 16 * hb, 16)]) * cs
                      for hb in range(4)]
                for b, j in enumerate(blocks):
                    a_v = stages[b]
                    if j == 0:
                        for hb in range(4):
                            d = a_v[e, pl.ds(16 * hb, 16)] * zs[hb]
                            pay_v[e, pl.ds(64 * b + 16 * hb, 16)] = d
                    else:
                        g = grow[j - 1]
                        for hb in range(4):
                            d = a_v[e, pl.ds(16 * hb, 16)] * zs[hb]
                            pay_v[e, pl.ds(64 * b + 16 * hb, 16)] = d * g
                return 0
            lax.fori_loop(0, C, edge, 0, unroll=False)
            pltpu.async_copy(pay_v, acc.at[src_b[b]], semC, add=True)

        def pair(k2, _):
            for b in range(2):
                chunk_b(k_lo + 2 * k2 + b, b)
            return 0
        lax.fori_loop(0, (k_hi - k_lo) // 2, pair, 0, unroll=False)
        pltpu.make_async_copy(pay_b[1], acc.at[src_b[1]], semC).wait()

    for p in range(3):
        pltpu.sync_copy(zr_hbm, acc.at[pl.ds(sid * ROWS_PT, ROWS_PT)])
        plsc.subcore_barrier()
        if p < 2:
            bl0, bl1 = SLOT_BLOCKS[2 * p], SLOT_BLOCKS[2 * p + 1]

            @pl.when(core == 0)
            def _c0():
                run_chunks(0, NCHUNK, bl0)

            @pl.when(core == 1)
            def _c1():
                run_chunks(0, NCHUNK, bl1)
        else:
            bl = SLOT_BLOCKS[4]

            @pl.when(core == 0)
            def _c0():
                run_chunks(0, SPLIT0, bl)

            @pl.when(core == 1)
            def _c1():
                run_chunks(SPLIT0, NCHUNK, bl)

        plsc.subcore_barrier()
        pltpu.sync_copy(acc.at[pl.ds(sid * ROWS_PT, ROWS_PT)],
                        w_hbm.at[2 * p + core, pl.ds(sid * ROWS_PT, ROWS_PT)])
        plsc.subcore_barrier()


NB3 = 2000


def _final_body(w_ref, lng_ref, lnb_ref, ls0_ref, ls0b_ref,
                ls1a_ref, ls1b_ref, ls1c_ref, ls1bias_ref,
                lt0_ref, lt1_ref, lt2_ref, o_ref):
    w = w_ref[...]
    s1 = w[0, :, 0:H]
    w2 = [w[0, :, H:2 * H], w[1, :, 0:H], w[1, :, H:2 * H]]
    w3 = [w[2, :, 0:H], w[2, :, H:2 * H],
          w[3, :, 0:H], w[3, :, H:2 * H],
          w[4, :, 0:H] + w[5, :, 0:H], w[4, :, H:2 * H] + w[5, :, H:2 * H]]
    nrm = (3.0 * s1 * s1
           + 2.0 * (w2[0] * w2[0] + w2[1] * w2[1] + w2[2] * w2[2])
           + w3[0] * w3[0] + w3[1] * w3[1] + w3[2] * w3[2]
           + 2.0 * (w3[3] * w3[3] + w3[4] * w3[4] + w3[5] * w3[5]))
    nrm = jnp.maximum(nrm, 0.01)
    mu = jnp.mean(nrm, axis=-1, keepdims=True)
    var = jnp.mean((nrm - mu) ** 2, axis=-1, keepdims=True)
    nrm = (nrm - mu) / jnp.sqrt(var + 1e-5) * lng_ref[...] + lnb_ref[...]
    f = _silu(jnp.dot(nrm, ls0_ref[...], preferred_element_type=jnp.float32)
              + ls0b_ref[...])
    n0 = _silu(jnp.dot(f, ls1a_ref[...], preferred_element_type=jnp.float32)
               + ls1bias_ref[0, :][None, :])
    n1 = _silu(jnp.dot(f, ls1b_ref[...], preferred_element_type=jnp.float32)
               + ls1bias_ref[1, :][None, :])
    n2 = _silu(jnp.dot(f, ls1c_ref[...], preferred_element_type=jnp.float32)
               + ls1bias_ref[2, :][None, :])
    ip = jnp.dot(s1, lt0_ref[...], preferred_element_type=jnp.float32) * n0
    w2p = [jnp.dot(w2[i], lt1_ref[...], preferred_element_type=jnp.float32) * n1
           for i in range(3)]
    w3p = [jnp.dot(w3[i], lt2_ref[...], preferred_element_type=jnp.float32) * n2
           for i in range(6)]
    o_ref[0] = ip + w3p[0]
    o_ref[1] = -w2p[2] + w3p[3]
    o_ref[2] = w2p[1] + w3p[4]
    o_ref[3] = w2p[2] + w3p[3]
    o_ref[4] = ip + w3p[1]
    o_ref[5] = -w2p[0] + w3p[5]
    o_ref[6] = -w2p[1] + w3p[4]
    o_ref[7] = w2p[0] + w3p[5]
    o_ref[8] = ip + w3p[2]


def _node_final(w, ln_g, ln_b, ls0_W, ls0_b, ls1_W, ls1_b, lt0_W, lt1_W, lt2_W):
    ls1bias = jnp.stack([ls1_b[0::3], ls1_b[1::3], ls1_b[2::3]], axis=0)
    full = lambda shape: pl.BlockSpec(shape, lambda i: (0,) * len(shape))
    out = pl.pallas_call(
        _final_body,
        grid=(N // NB3,),
        in_specs=[
            pl.BlockSpec((6, NB3, 128), lambda i: (0, i, 0)),
            full((1, H)), full((1, H)),
            full((H, 2 * H)), full((1, 2 * H)),
            full((2 * H, H)), full((2 * H, H)), full((2 * H, H)), full((3, H)),
            full((H, H)), full((H, H)), full((H, H)),
        ],
        out_specs=pl.BlockSpec((9, NB3, H), lambda i: (0, i, 0)),
        out_shape=jax.ShapeDtypeStruct((9, N, H), jnp.float32),
    )(w, ln_g[None, :], ln_b[None, :], ls0_W.T, ls0_b[None, :],
      ls1_W[0::3, :].T, ls1_W[1::3, :].T, ls1_W[2::3, :].T, ls1bias,
      lt0_W.T, lt1_W.T, lt2_W.T)
    return out


@jax.jit
def kernel(z, t, edge_index, edge_weight, edge_vec_norm, edge_attr, node_attr,
           emb_table, mix1_W, mix1_b, mix2_W, mix2_b, emb2_W, emb2_b,
           dp1_W, dp1_b, dp2_W, dp2_b, dp3_W, dp3_b,
           lt0_W, lt1_W, lt2_W, ls0_W, ls0_b, ls1_W, ls1_b, ln_g, ln_b):
    pq = _node_prep(z, t, emb_table, mix1_W, mix1_b, mix2_W, mix2_b,
                    emb2_W, emb2_b)
    a1, a2, a3 = _edge_coef(edge_attr, dp1_W, dp1_b, dp2_W, dp2_b,
                            dp3_W, dp3_b)
    g = _geom(edge_weight, edge_vec_norm)
    zr = jnp.zeros((ROWS_PT, 128), jnp.float32)
    ei = edge_index.astype(jnp.int32)
    w = _sc_edge(ei[0], ei[1], pq, a1, a2, a3, g, zr)
    planes = _node_final(w, ln_g, ln_b, ls0_W, ls0_b, ls1_W, ls1_b,
                         lt0_W, lt1_W, lt2_W)
    return planes.transpose(1, 2, 0).reshape(N, H, 3, 3)
